# baseline (speedup 1.0000x reference)
import functools

import jax
import jax.numpy as jnp
from jax import lax
from jax.experimental import pallas as pl
from jax.experimental.pallas import tpu as pltpu
from jax.experimental.pallas import tpu_sc as plsc

V, M, N = 50000, 512, 16384
NC, NS, LANES = 2, 16, 16
NW = NC * NS
B_PER_W = N // NW
CHUNK = 64
NCHUNK = B_PER_W // CHUNK
MV = M // LANES


def _sc_body(eig_hbm, table_hbm, idx_hbm, ls_hbm, var_hbm, out_hbm,
             idx_v, s_v, eig_v, ls_v, var_v, bufs, gsem0, gsem1):
    cid = lax.axis_index("c")
    sid = lax.axis_index("s")
    wid = sid * NC + cid
    base = wid * B_PER_W

    pltpu.sync_copy(idx_hbm.at[wid], idx_v)
    pltpu.sync_copy(eig_hbm, eig_v)
    pltpu.sync_copy(ls_hbm, ls_v)
    pltpu.sync_copy(var_hbm, var_v)

    ls = ls_v[...]
    var = var_v[...]
    coef = -0.5 / (ls * ls)
    for j in range(MV):
        sl = pl.ds(j * LANES, LANES)
        s_v[sl] = var * jnp.exp(eig_v[sl] * coef)

    gsems = (gsem0, gsem1)

    def scale(b):
        def row_body(r, carry):
            for j in range(MV):
                sl = pl.ds(j * LANES, LANES)
                bufs[b, r, sl] = bufs[b, r, sl] * s_v[sl]
            return carry
        lax.fori_loop(0, CHUNK, row_body, 0)

    copies = [None] * NCHUNK
    copies[0] = pltpu.async_copy(table_hbm.at[idx_v.at[0]], bufs.at[0], gsems[0])
    for c in range(NCHUNK):
        b = c % 2
        if c + 1 < NCHUNK:
            nb = (c + 1) % 2
            copies[c + 1] = pltpu.async_copy(
                table_hbm.at[idx_v.at[c + 1]], bufs.at[nb], gsems[nb])
        copies[c].wait()
        scale(b)
        pltpu.sync_copy(bufs.at[b], out_hbm.at[pl.ds(base + c * CHUNK, CHUNK)])


_mesh = plsc.VectorSubcoreMesh(core_axis_name="c", subcore_axis_name="s")

_sc_kernel = functools.partial(
    pl.kernel,
    mesh=_mesh,
    out_type=jax.ShapeDtypeStruct((N, M), jnp.float32),
    scratch_types=[
        pltpu.VMEM((NCHUNK, CHUNK), jnp.int32),
        pltpu.VMEM((M,), jnp.float32),
        pltpu.VMEM((M,), jnp.float32),
        pltpu.VMEM((LANES,), jnp.float32),
        pltpu.VMEM((LANES,), jnp.float32),
        pltpu.VMEM((2, CHUNK, M), jnp.float32),
        pltpu.SemaphoreType.DMA,
        pltpu.SemaphoreType.DMA,
    ],
)(_sc_body)


def kernel(eigvals, eigvecs, node_indices, lengthscale, variance):
    idx = node_indices.astype(jnp.int32).reshape(NW, NCHUNK, CHUNK)
    ls = jnp.broadcast_to(lengthscale.astype(jnp.float32), (LANES,))
    var = jnp.broadcast_to(variance.astype(jnp.float32), (LANES,))
    return _sc_kernel(eigvals, eigvecs, idx, ls, var)


# speedup vs baseline: 2.3604x; 2.3604x over previous
import functools

import jax
import jax.numpy as jnp
from jax import lax
from jax.experimental import pallas as pl
from jax.experimental.pallas import tpu as pltpu
from jax.experimental.pallas import tpu_sc as plsc

V, M, N = 50000, 512, 16384
NC, NS, LANES = 2, 16, 16
NW = NC * NS
B_PER_W = N // NW
CHUNK = 64
NCHUNK = B_PER_W // CHUNK
MV = M // LANES


NBUF = 3


def _sc_body(eig_hbm, table_hbm, idx_hbm, ls_hbm, var_hbm, out_hbm,
             idx_v, s_v, eig_v, ls_v, var_v, bufs,
             gsem0, gsem1, gsem2, ssem0, ssem1, ssem2):
    cid = lax.axis_index("c")
    sid = lax.axis_index("s")
    wid = sid * NC + cid
    base = wid * B_PER_W

    pltpu.sync_copy(idx_hbm.at[wid], idx_v)
    pltpu.sync_copy(eig_hbm, eig_v)
    pltpu.sync_copy(ls_hbm, ls_v)
    pltpu.sync_copy(var_hbm, var_v)

    ls = ls_v[...]
    var = var_v[...]
    coef = -0.5 / (ls * ls)
    for j in range(MV):
        sl = pl.ds(j * LANES, LANES)
        s_v[sl] = var * jnp.exp(eig_v[sl] * coef)

    gsems = (gsem0, gsem1, gsem2)
    ssems = (ssem0, ssem1, ssem2)

    s_regs = [s_v[pl.ds(j * LANES, LANES)] for j in range(MV)]

    def scale(b):
        def row_body(r, carry):
            for j in range(MV):
                sl = pl.ds(j * LANES, LANES)
                bufs[b, r, sl] = bufs[b, r, sl] * s_regs[j]
            return carry
        lax.fori_loop(0, CHUNK, row_body, 0)

    def gather(c):
        return pltpu.async_copy(table_hbm.at[idx_v.at[c]], bufs.at[c % NBUF],
                                gsems[c % NBUF])

    gcopies = [None] * NCHUNK
    scopies = [None] * NCHUNK
    gcopies[0] = gather(0)
    gcopies[1] = gather(1)
    for c in range(NCHUNK):
        b = c % NBUF
        gcopies[c].wait()
        scale(b)
        scopies[c] = pltpu.async_copy(
            bufs.at[b], out_hbm.at[pl.ds(base + c * CHUNK, CHUNK)], ssems[b])
        if c + 2 < NCHUNK:
            if c - 1 >= 0:
                scopies[c - 1].wait()
            gcopies[c + 2] = gather(c + 2)
    scopies[NCHUNK - 2].wait()
    scopies[NCHUNK - 1].wait()


_mesh = plsc.VectorSubcoreMesh(core_axis_name="c", subcore_axis_name="s")

_sc_kernel = functools.partial(
    pl.kernel,
    mesh=_mesh,
    out_type=jax.ShapeDtypeStruct((N, M), jnp.float32),
    scratch_types=[
        pltpu.VMEM((NCHUNK, CHUNK), jnp.int32),
        pltpu.VMEM((M,), jnp.float32),
        pltpu.VMEM((M,), jnp.float32),
        pltpu.VMEM((LANES,), jnp.float32),
        pltpu.VMEM((LANES,), jnp.float32),
        pltpu.VMEM((NBUF, CHUNK, M), jnp.float32),
        pltpu.SemaphoreType.DMA,
        pltpu.SemaphoreType.DMA,
        pltpu.SemaphoreType.DMA,
        pltpu.SemaphoreType.DMA,
        pltpu.SemaphoreType.DMA,
        pltpu.SemaphoreType.DMA,
    ],
)(_sc_body)


def kernel(eigvals, eigvecs, node_indices, lengthscale, variance):
    idx = node_indices.astype(jnp.int32).reshape(NW, NCHUNK, CHUNK)
    ls = jnp.broadcast_to(lengthscale.astype(jnp.float32), (LANES,))
    var = jnp.broadcast_to(variance.astype(jnp.float32), (LANES,))
    return _sc_kernel(eigvals, eigvecs, idx, ls, var)


# speedup vs baseline: 2.4208x; 1.0256x over previous
import functools

import jax
import jax.numpy as jnp
from jax import lax
from jax.experimental import pallas as pl
from jax.experimental.pallas import tpu as pltpu
from jax.experimental.pallas import tpu_sc as plsc

V, M, N = 50000, 512, 16384
NC, NS, LANES = 2, 16, 16
NW = NC * NS
B_PER_W = N // NW
CHUNK = 64
NCHUNK = B_PER_W // CHUNK
MV = M // LANES


NBUF = 3


def _sc_body(eig_hbm, table_hbm, idx_hbm, ls_hbm, var_hbm, out_hbm,
             idx_v, s_v, eig_v, ls_v, var_v, bufs,
             gsem0, gsem1, gsem2, ssem0, ssem1, ssem2):
    cid = lax.axis_index("c")
    sid = lax.axis_index("s")
    wid = sid * NC + cid
    base = wid * B_PER_W

    gsems = (gsem0, gsem1, gsem2)
    ssems = (ssem0, ssem1, ssem2)

    pltpu.sync_copy(idx_hbm.at[wid], idx_v)
    first = pltpu.async_copy(table_hbm.at[idx_v.at[0]], bufs.at[0], gsems[0])
    second = pltpu.async_copy(table_hbm.at[idx_v.at[1]], bufs.at[1], gsems[1])

    pltpu.sync_copy(eig_hbm, eig_v)
    pltpu.sync_copy(ls_hbm, ls_v)
    pltpu.sync_copy(var_hbm, var_v)

    ls = ls_v[...]
    var = var_v[...]
    coef = -0.5 / (ls * ls)
    for j in range(MV):
        sl = pl.ds(j * LANES, LANES)
        s_v[sl] = var * jnp.exp(eig_v[sl] * coef)

    s_regs = [s_v[pl.ds(j * LANES, LANES)] for j in range(MV)]

    def scale(b):
        def row_body(r, carry):
            for j in range(MV):
                sl = pl.ds(j * LANES, LANES)
                bufs[b, r, sl] = bufs[b, r, sl] * s_regs[j]
            return carry
        lax.fori_loop(0, CHUNK, row_body, 0)

    def gather(c):
        return pltpu.async_copy(table_hbm.at[idx_v.at[c]], bufs.at[c % NBUF],
                                gsems[c % NBUF])

    gcopies = [None] * NCHUNK
    scopies = [None] * NCHUNK
    gcopies[0] = first
    gcopies[1] = second
    for c in range(NCHUNK):
        b = c % NBUF
        gcopies[c].wait()
        scale(b)
        scopies[c] = pltpu.async_copy(
            bufs.at[b], out_hbm.at[pl.ds(base + c * CHUNK, CHUNK)], ssems[b])
        if c + 2 < NCHUNK:
            if c - 1 >= 0:
                scopies[c - 1].wait()
            gcopies[c + 2] = gather(c + 2)
    scopies[NCHUNK - 2].wait()
    scopies[NCHUNK - 1].wait()


_mesh = plsc.VectorSubcoreMesh(core_axis_name="c", subcore_axis_name="s")

_sc_kernel = functools.partial(
    pl.kernel,
    mesh=_mesh,
    out_type=jax.ShapeDtypeStruct((N, M), jnp.float32),
    scratch_types=[
        pltpu.VMEM((NCHUNK, CHUNK), jnp.int32),
        pltpu.VMEM((M,), jnp.float32),
        pltpu.VMEM((M,), jnp.float32),
        pltpu.VMEM((LANES,), jnp.float32),
        pltpu.VMEM((LANES,), jnp.float32),
        pltpu.VMEM((NBUF, CHUNK, M), jnp.float32),
        pltpu.SemaphoreType.DMA,
        pltpu.SemaphoreType.DMA,
        pltpu.SemaphoreType.DMA,
        pltpu.SemaphoreType.DMA,
        pltpu.SemaphoreType.DMA,
        pltpu.SemaphoreType.DMA,
    ],
)(_sc_body)


def kernel(eigvals, eigvecs, node_indices, lengthscale, variance):
    idx = node_indices.astype(jnp.int32).reshape(NW, NCHUNK, CHUNK)
    ls = jnp.broadcast_to(lengthscale.astype(jnp.float32), (LANES,))
    var = jnp.broadcast_to(variance.astype(jnp.float32), (LANES,))
    return _sc_kernel(eigvals, eigvecs, idx, ls, var)
